# baseline (speedup 1.0000x reference)
import functools

import jax
import jax.numpy as jnp
from jax import lax
from jax.experimental import pallas as pl
from jax.experimental.pallas import tpu as pltpu
from jax.experimental.pallas import tpu_sc as plsc

BATCH = 1024
LENGTH = 50
DIM = 32
BINS = DIM + DIM * DIM

NUM_CORES = 2
NUM_SUBCORES = 16
LANES = 16
STRIPES = 8
QUARTERS = 4
BINS_PER_W = BINS // QUARTERS
HALF_A = 136
HALF_B = BINS_PER_W - HALF_A
COLS_PER_W = BATCH // STRIPES
GROUPS = COLS_PER_W // LANES
PAIRS = LENGTH // 2


@functools.partial(
    pl.kernel,
    out_type=jax.ShapeDtypeStruct((BINS, BATCH), jnp.float32),
    mesh=plsc.VectorSubcoreMesh(core_axis_name="c", subcore_axis_name="s"),
    scratch_types=[
        pltpu.VMEM((LENGTH, COLS_PER_W), jnp.int32),
        pltpu.VMEM((HALF_A, COLS_PER_W), jnp.float32),
        pltpu.VMEM((HALF_B, COLS_PER_W), jnp.float32),
        pltpu.SemaphoreType.DMA,
    ],
    compiler_params=pltpu.CompilerParams(
        needs_layout_passes=False, disable_bounds_checks=True
    ),
)
def _ngram_counts_sc(in_hbm, out_hbm, tok_v, cnt_a, cnt_b, sem):
    wid = lax.axis_index("s") * NUM_CORES + lax.axis_index("c")
    stripe = lax.rem(wid, STRIPES)
    quarter = lax.div(wid, STRIPES)
    col0 = stripe * COLS_PER_W
    lo = quarter * BINS_PER_W

    pltpu.sync_copy(in_hbm.at[:, pl.ds(col0, COLS_PER_W)], tok_v)

    zeros = jnp.zeros((LANES,), jnp.float32)
    lane = lax.iota(jnp.int32, 16)
    ones = jnp.ones((LANES,), jnp.float32)

    def _zero(ref, rows):
        def _zero_body(i, carry):
            for j in range(2):
                for c in range(COLS_PER_W // LANES):
                    ref[i * 2 + j, pl.ds(c * LANES, LANES)] = zeros
            return carry

        lax.fori_loop(0, rows // 2, _zero_body, 0)

    def _scatter(ref, bin_v, base_v, rows, col):
        local = bin_v - base_v
        mask = plsc.bitcast(local, jnp.uint32) < jnp.full(
            (LANES,), rows, jnp.uint32
        )
        plsc.addupdate_scatter(ref, [local, col], ones, mask=mask)

    _zero(cnt_a, HALF_A)
    lo_v = jnp.full((LANES,), 0, jnp.int32) + lo

    def _pass1_body(g, carry):
        coff = g * LANES
        col = coff + lane
        for p in range(PAIRS):
            a = tok_v[2 * p, pl.ds(coff, LANES)]
            b = tok_v[2 * p + 1, pl.ds(coff, LANES)]
            _scatter(cnt_a, a, lo_v, HALF_A, col)
            _scatter(cnt_a, b, lo_v, HALF_A, col)
            _scatter(cnt_a, DIM + a * DIM + b, lo_v, HALF_A, col)
        return carry

    lax.fori_loop(0, GROUPS, _pass1_body, 0)
    flush_a = pltpu.async_copy(
        cnt_a, out_hbm.at[pl.ds(lo, HALF_A), pl.ds(col0, COLS_PER_W)], sem
    )

    _zero(cnt_b, HALF_B)
    mid_v = jnp.full((LANES,), 0, jnp.int32) + (lo + HALF_A)

    def _pass2_body(g, carry):
        coff = g * LANES
        col = coff + lane
        for p in range(PAIRS):
            a = tok_v[2 * p, pl.ds(coff, LANES)]
            b = tok_v[2 * p + 1, pl.ds(coff, LANES)]
            _scatter(cnt_b, DIM + a * DIM + b, mid_v, HALF_B, col)
        return carry

    lax.fori_loop(0, GROUPS, _pass2_body, 0)
    flush_a.wait()
    pltpu.sync_copy(
        cnt_b, out_hbm.at[pl.ds(lo + HALF_A, HALF_B), pl.ds(col0, COLS_PER_W)]
    )


def kernel(inputs):
    out_t = _ngram_counts_sc(inputs.T)
    return out_t.T


# speedup vs baseline: 1.0471x; 1.0471x over previous
import functools

import jax
import jax.numpy as jnp
from jax import lax
from jax.experimental import pallas as pl
from jax.experimental.pallas import tpu as pltpu
from jax.experimental.pallas import tpu_sc as plsc

BATCH = 1024
LENGTH = 50
DIM = 32
BINS = DIM + DIM * DIM

NUM_CORES = 2
NUM_SUBCORES = 16
LANES = 16
STRIPES = 8
QUARTERS = 4
BINS_PER_W = BINS // QUARTERS
HALF_A = 136
HALF_B = BINS_PER_W - HALF_A
COLS_PER_W = BATCH // STRIPES
GROUPS = COLS_PER_W // LANES
PAIRS = LENGTH // 2


@functools.partial(
    pl.kernel,
    out_type=jax.ShapeDtypeStruct((BINS, BATCH), jnp.float32),
    mesh=plsc.VectorSubcoreMesh(core_axis_name="c", subcore_axis_name="s"),
    scratch_types=[
        pltpu.VMEM((LENGTH, COLS_PER_W), jnp.int32),
        pltpu.VMEM((HALF_A, COLS_PER_W), jnp.float32),
        pltpu.VMEM((HALF_B, COLS_PER_W), jnp.float32),
        pltpu.VMEM((GROUPS * PAIRS, LANES), jnp.int32),
        pltpu.SemaphoreType.DMA,
        pltpu.SemaphoreType.DMA,
    ],
    compiler_params=pltpu.CompilerParams(
        needs_layout_passes=False, disable_bounds_checks=True
    ),
)
def _ngram_counts_sc(in_hbm, out_hbm, tok_v, cnt_a, cnt_b, big_v, sem_in, sem_a):
    wid = lax.axis_index("s") * NUM_CORES + lax.axis_index("c")
    stripe = lax.rem(wid, STRIPES)
    quarter = lax.div(wid, STRIPES)
    col0 = stripe * COLS_PER_W
    lo = quarter * BINS_PER_W

    stage = pltpu.async_copy(
        in_hbm.at[:, pl.ds(col0, COLS_PER_W)], tok_v, sem_in
    )

    zeros = jnp.zeros((LANES,), jnp.float32)
    lane = lax.iota(jnp.int32, 16)
    ones = jnp.ones((LANES,), jnp.float32)

    def _zero(ref, rows):
        def _zero_body(i, carry):
            for j in range(2):
                for c in range(COLS_PER_W // LANES):
                    ref[i * 2 + j, pl.ds(c * LANES, LANES)] = zeros
            return carry

        lax.fori_loop(0, rows // 2, _zero_body, 0)

    def _scatter(ref, bin_v, base_v, rows, col):
        local = bin_v - base_v
        mask = plsc.bitcast(local, jnp.uint32) < jnp.full(
            (LANES,), rows, jnp.uint32
        )
        plsc.addupdate_scatter(ref, [local, col], ones, mask=mask)

    _zero(cnt_a, HALF_A)
    stage.wait()
    lo_v = jnp.full((LANES,), 0, jnp.int32) + lo

    def _pass1_body(g, carry):
        coff = g * LANES
        col = coff + lane
        for p in range(PAIRS):
            a = tok_v[2 * p, pl.ds(coff, LANES)]
            b = tok_v[2 * p + 1, pl.ds(coff, LANES)]
            big = DIM + a * DIM + b
            big_v[g * PAIRS + p, :] = big
            _scatter(cnt_a, a, lo_v, HALF_A, col)
            _scatter(cnt_a, b, lo_v, HALF_A, col)
            _scatter(cnt_a, big, lo_v, HALF_A, col)
        return carry

    lax.fori_loop(0, GROUPS, _pass1_body, 0)
    flush_a = pltpu.async_copy(
        cnt_a, out_hbm.at[pl.ds(lo, HALF_A), pl.ds(col0, COLS_PER_W)], sem_a
    )

    _zero(cnt_b, HALF_B)
    mid_v = jnp.full((LANES,), 0, jnp.int32) + (lo + HALF_A)

    def _pass2_body(g, carry):
        coff = g * LANES
        col = coff + lane
        for p in range(PAIRS):
            big = big_v[g * PAIRS + p, :]
            _scatter(cnt_b, big, mid_v, HALF_B, col)
        return carry

    lax.fori_loop(0, GROUPS, _pass2_body, 0)
    flush_a.wait()
    pltpu.sync_copy(
        cnt_b, out_hbm.at[pl.ds(lo + HALF_A, HALF_B), pl.ds(col0, COLS_PER_W)]
    )


def kernel(inputs):
    out_t = _ngram_counts_sc(inputs.T)
    return out_t.T
